# Initial kernel scaffold; baseline (speedup 1.0000x reference)
import jax
import jax.numpy as jnp
from jax.experimental import pallas as pl


def kernel(x, edge_index, edge_attr, Wq0, bq0, Wk0, bk0, Wv0, bv0, Ws0, bs0, Wq1, bq1, Wk1, bk1, Wv1, bv1, Ws1, bs1, Wl, bl):
    raise NotImplementedError("write your pallas kernel here")


# speedup vs baseline: 2.2476x; 2.2476x over previous
import functools

import jax
import jax.numpy as jnp
from jax import lax
from jax.experimental import pallas as pl
from jax.experimental.pallas import tpu as pltpu
from jax.experimental.pallas import tpu_sc as plsc

N = 10000
E = 320000
H = 4
C = 128
D = H * C

NC = 2
NS = 16
NW = NC * NS
EPW = E // NW
CH = 80
NCHUNK = EPW // CH
G = CH // 16
SLAB = 640
NPAD = NS * SLAB

_SCALE = 1.0 / float(C) ** 0.5


_BN = 1000
_NBLK = N // _BN


def _proj_body(x_ref, wq_ref, bq_ref, wk_ref, bk_ref, wv_ref, bv_ref,
               ws_ref, bs_ref, q_ref, k_ref, v_ref, s_ref):
    x = x_ref[...]
    q = jnp.dot(x, wq_ref[...], preferred_element_type=jnp.float32) + bq_ref[...]
    q_ref[...] = q * _SCALE
    k_ref[...] = jnp.dot(x, wk_ref[...], preferred_element_type=jnp.float32) + bk_ref[...]
    v_ref[...] = jnp.dot(x, wv_ref[...], preferred_element_type=jnp.float32) + bv_ref[...]
    s_ref[...] = jnp.dot(x, ws_ref[...], preferred_element_type=jnp.float32) + bs_ref[...]


def _project(x, Wq, bq, Wk, bk, Wv, bv, Ws, bs):
    din = x.shape[1]
    wspec = pl.BlockSpec((din, D), lambda i: (0, 0))
    bspec = pl.BlockSpec((1, D), lambda i: (0, 0))
    ospec = pl.BlockSpec((_BN, D), lambda i: (i, 0))
    out_sh = jax.ShapeDtypeStruct((N, D), jnp.float32)
    return pl.pallas_call(
        _proj_body,
        grid=(_NBLK,),
        in_specs=[pl.BlockSpec((_BN, din), lambda i: (i, 0)),
                  wspec, bspec, wspec, bspec, wspec, bspec, wspec, bspec],
        out_specs=[ospec, ospec, ospec, ospec],
        out_shape=[out_sh, out_sh, out_sh, out_sh],
    )(x, Wq, bq.reshape(1, D), Wk, bk.reshape(1, D),
      Wv, bv.reshape(1, D), Ws, bs.reshape(1, D))


_MESH = plsc.VectorSubcoreMesh(core_axis_name="c", subcore_axis_name="s")


@functools.partial(
    pl.kernel,
    out_type=(jax.ShapeDtypeStruct((H * E,), jnp.float32),
              jax.ShapeDtypeStruct((NW * 16,), jnp.float32)),
    mesh=_MESH,
    compiler_params=pltpu.CompilerParams(use_tc_tiling_on_sc=False, needs_layout_passes=False),
    scratch_types=[
        pltpu.VMEM((CH,), jnp.int32),
        pltpu.VMEM((CH,), jnp.int32),
        pltpu.VMEM((CH, D), jnp.float32),
        pltpu.VMEM((CH, D), jnp.float32),
        pltpu.VMEM((H * CH,), jnp.float32),
        pltpu.VMEM((16,), jnp.float32),
        pltpu.SemaphoreType.DMA,
        pltpu.SemaphoreType.DMA,
    ],
)
def _alpha_kernel(q_hbm, k_hbm, src_hbm, dst_hbm, alpha_hbm, tmax_hbm,
                  dstbuf, srcbuf, qrows, krows, abuf, maxref, semq, semk):
    cid = lax.axis_index("c")
    sid = lax.axis_index("s")
    wid = cid * NS + sid
    maxref[...] = jnp.full((16,), -jnp.inf, jnp.float32)
    lanes = lax.iota(jnp.int32, 16)

    def chunk_body(ch, carry):
        base = wid * EPW + ch * CH
        pltpu.sync_copy(dst_hbm.at[pl.ds(base, CH)], dstbuf)
        pltpu.sync_copy(src_hbm.at[pl.ds(base, CH)], srcbuf)
        cq = pltpu.async_copy(q_hbm.at[dstbuf], qrows, semq)
        ck = pltpu.async_copy(k_hbm.at[srcbuf], krows, semk)
        cq.wait()
        ck.wait()
        for g in range(G):
            eidx = lanes + g * 16
            for h in range(H):
                def cbody(c0, acc, _h=h, _eidx=eidx):
                    for cc in range(16):
                        col = jnp.full((16,), _h * C + c0 * 16 + cc, jnp.int32)
                        qv = plsc.load_gather(qrows, [_eidx, col])
                        kv = plsc.load_gather(krows, [_eidx, col])
                        acc = acc + qv * kv
                    return acc
                acc = lax.fori_loop(0, C // 16, cbody,
                                    jnp.zeros((16,), jnp.float32))
                abuf[pl.ds(h * CH + g * 16, 16)] = acc
                maxref[...] = jnp.maximum(maxref[...], acc)
        for h in range(H):
            pltpu.sync_copy(abuf.at[pl.ds(h * CH, CH)],
                            alpha_hbm.at[pl.ds(h * E + base, CH)])
        return carry

    lax.fori_loop(0, NCHUNK, chunk_body, 0)
    pltpu.sync_copy(maxref, tmax_hbm.at[pl.ds(wid * 16, 16)])


@functools.partial(
    pl.kernel,
    out_type=(jax.ShapeDtypeStruct((NC, NPAD, C), jnp.float32),
              jax.ShapeDtypeStruct((NC, NPAD, 16), jnp.float32)),
    mesh=_MESH,
    compiler_params=pltpu.CompilerParams(use_tc_tiling_on_sc=False, needs_layout_passes=False),
    scratch_types=[
        pltpu.VMEM((CH,), jnp.int32),
        pltpu.VMEM((CH,), jnp.int32),
        pltpu.VMEM((CH,), jnp.float32),
        pltpu.VMEM((CH, C), jnp.float32),
        pltpu.VMEM((CH, C), jnp.float32),
        pltpu.VMEM((CH, 16), jnp.float32),
        pltpu.VMEM((NW * 16,), jnp.float32),
        pltpu.VMEM_SHARED((NPAD, C), jnp.float32),
        pltpu.VMEM_SHARED((NPAD, 16), jnp.float32),
        pltpu.SemaphoreType.DMA,
    ],
)
def _accum_kernel(v_hbm, a_hbm, src_hbm, dst_hbm, tmax_hbm, znum_hbm, zden_hbm,
                  num_hbm, den_hbm,
                  srcbuf, dstbuf, abuf, vrows, wrows, dbuf, tmbuf,
                  spnum, spden, sem):
    cid = lax.axis_index("c")
    sid = lax.axis_index("s")
    wid = cid * NS + sid

    pltpu.sync_copy(tmax_hbm, tmbuf)
    m = tmbuf[pl.ds(0, 16)]
    for k in range(1, NW):
        m = jnp.maximum(m, tmbuf[pl.ds(k * 16, 16)])
    gmax = jnp.full((16,), jnp.max(m), jnp.float32)

    zero16 = jnp.zeros((16,), jnp.float32)
    for r in range(CH):
        dbuf[r] = zero16

    pltpu.sync_copy(znum_hbm, spnum.at[pl.ds(sid * SLAB, SLAB)])
    pltpu.sync_copy(zden_hbm, spden.at[pl.ds(sid * SLAB, SLAB)])
    plsc.subcore_barrier()

    lanes = lax.iota(jnp.int32, 16)
    zcol = jnp.zeros((16,), jnp.int32)

    def chunk_body(ch, carry):
        base = wid * EPW + ch * CH
        pltpu.sync_copy(dst_hbm.at[pl.ds(base, CH)], dstbuf)
        pltpu.sync_copy(src_hbm.at[pl.ds(base, CH)], srcbuf)
        pltpu.sync_copy(a_hbm.at[pl.ds(base, CH)], abuf)
        pltpu.async_copy(v_hbm.at[srcbuf], vrows, sem).wait()
        for g in range(G):
            eidx = lanes + g * 16
            a = abuf[pl.ds(g * 16, 16)]
            w = jnp.exp(a - gmax)
            plsc.store_scatter(dbuf, [eidx, zcol], w)

            def cbody(c0, carry2, _eidx=eidx, _w=w):
                for cc in range(16):
                    col = jnp.full((16,), c0 * 16 + cc, jnp.int32)
                    vv = plsc.load_gather(vrows, [_eidx, col])
                    plsc.store_scatter(wrows, [_eidx, col], vv * _w)
                return carry2
            lax.fori_loop(0, C // 16, cbody, 0)
        pltpu.sync_copy(wrows, spnum.at[dstbuf], add=True)
        pltpu.sync_copy(dbuf, spden.at[dstbuf], add=True)
        return carry

    lax.fori_loop(0, NCHUNK, chunk_body, 0)
    plsc.subcore_barrier()
    pltpu.sync_copy(spnum.at[pl.ds(sid * SLAB, SLAB)],
                    num_hbm.at[cid, pl.ds(sid * SLAB, SLAB)])
    pltpu.sync_copy(spden.at[pl.ds(sid * SLAB, SLAB)],
                    den_hbm.at[cid, pl.ds(sid * SLAB, SLAB)])


def _merge_heads(nrefs, drefs, s_ref):
    outs = []
    for nh, dh in zip(nrefs, drefs):
        n = nh[0] + nh[1]
        d = dh[0, :, 0:1] + dh[1, :, 0:1]
        safe = jnp.where(d > 0, d, 1.0)
        outs.append(jnp.where(d > 0, n / safe, 0.0))
    h = jnp.concatenate(outs, axis=-1) + s_ref[...]
    return jnp.maximum(h, 0.0)


def _combine_body(n0, n1, n2, n3, d0, d1, d2, d3, s_ref, o_ref):
    o_ref[...] = _merge_heads((n0, n1, n2, n3), (d0, d1, d2, d3), s_ref)


def _combine_final_body(n0, n1, n2, n3, d0, d1, d2, d3, s_ref, wl_ref, bl_ref,
                        o_ref):
    h = _merge_heads((n0, n1, n2, n3), (d0, d1, d2, d3), s_ref)
    z = jnp.dot(h, wl_ref[...], preferred_element_type=jnp.float32) + bl_ref[...]
    o_ref[...] = jax.nn.sigmoid(z)


_NSPEC = pl.BlockSpec((NC, _BN, C), lambda i: (0, i, 0))
_DSPEC = pl.BlockSpec((NC, _BN, 16), lambda i: (0, i, 0))
_SSPEC = pl.BlockSpec((_BN, D), lambda i: (i, 0))


def _combine(nums, dens, S):
    return pl.pallas_call(
        _combine_body,
        grid=(_NBLK,),
        in_specs=[_NSPEC] * 4 + [_DSPEC] * 4 + [_SSPEC],
        out_specs=_SSPEC,
        out_shape=jax.ShapeDtypeStruct((N, D), jnp.float32),
    )(*nums, *dens, S)


def _combine_final(nums, dens, S, Wl, bl):
    return pl.pallas_call(
        _combine_final_body,
        grid=(_NBLK,),
        in_specs=[_NSPEC] * 4 + [_DSPEC] * 4 + [_SSPEC,
                  pl.BlockSpec((D, 1), lambda i: (0, 0)),
                  pl.BlockSpec((1, 1), lambda i: (0, 0))],
        out_specs=pl.BlockSpec((_BN, 1), lambda i: (i, 0)),
        out_shape=jax.ShapeDtypeStruct((N, 1), jnp.float32),
    )(*nums, *dens, S, Wl, bl.reshape(1, 1))


def _attention_layer(xin, src, dst, znum, zden, Wq, bq, Wk, bk, Wv, bv, Ws, bs):
    Q, K, V, S = _project(xin, Wq, bq, Wk, bk, Wv, bv, Ws, bs)
    alpha, tmax = _alpha_kernel(Q, K, src, dst)
    Vh = V.reshape(N, H, C).transpose(1, 0, 2)
    nums, dens = [], []
    for h in range(H):
        num, den = _accum_kernel(Vh[h], alpha[h * E:(h + 1) * E], src, dst,
                                 tmax, znum, zden)
        nums.append(num)
        dens.append(den)
    return nums, dens, S


def kernel(x, edge_index, edge_attr, Wq0, bq0, Wk0, bk0, Wv0, bv0, Ws0, bs0,
           Wq1, bq1, Wk1, bk1, Wv1, bv1, Ws1, bs1, Wl, bl):
    src = edge_index[0]
    dst = edge_index[1]
    znum = jnp.zeros((SLAB, C), jnp.float32)
    zden = jnp.zeros((SLAB, 16), jnp.float32)

    nums, dens, S0 = _attention_layer(x, src, dst, znum, zden,
                                      Wq0, bq0, Wk0, bk0, Wv0, bv0, Ws0, bs0)
    h0 = _combine(nums, dens, S0)
    nums, dens, S1 = _attention_layer(h0, src, dst, znum, zden,
                                      Wq1, bq1, Wk1, bk1, Wv1, bv1, Ws1, bs1)
    return _combine_final(nums, dens, S1, Wl, bl)


# speedup vs baseline: 2.4928x; 1.1091x over previous
import functools

import jax
import jax.numpy as jnp
from jax import lax
from jax.experimental import pallas as pl
from jax.experimental.pallas import tpu as pltpu
from jax.experimental.pallas import tpu_sc as plsc

N = 10000
E = 320000
H = 4
C = 128
D = H * C
CHALF = C // 2

NC = 2
NS = 16
NW = NC * NS
EPW = E // NW
CH = 80
NCHUNK = EPW // CH
G = CH // 16
HCH = H * CH
EPS = E // NS
ACHUNK = EPS // CH
SLAB = 640
NPAD = NS * SLAB

_SCALE = 1.0 / float(C) ** 0.5
_SC_PARAMS = pltpu.CompilerParams(use_tc_tiling_on_sc=False,
                                  needs_layout_passes=False)


_BN = 1280
_NBLK = NPAD // _BN


def _proj_body(x_ref, wq_ref, bq_ref, wk_ref, bk_ref, wv_ref, bv_ref,
               ws_ref, bs_ref, q_ref, k_ref, v_ref, s_ref):
    x = x_ref[...]
    q = jnp.dot(x, wq_ref[...], preferred_element_type=jnp.float32) + bq_ref[...]
    q_ref[...] = q * _SCALE
    k_ref[...] = jnp.dot(x, wk_ref[...], preferred_element_type=jnp.float32) + bk_ref[...]
    v_ref[...] = jnp.dot(x, wv_ref[...], preferred_element_type=jnp.float32) + bv_ref[...]
    s_ref[...] = jnp.dot(x, ws_ref[...], preferred_element_type=jnp.float32) + bs_ref[...]


def _project(x, Wq, bq, Wk, bk, Wv, bv, Ws, bs):
    din = x.shape[1]
    wspec = pl.BlockSpec((din, D), lambda i: (0, 0))
    bspec = pl.BlockSpec((1, D), lambda i: (0, 0))
    ospec = pl.BlockSpec((_BN, D), lambda i: (i, 0))
    out_sh = jax.ShapeDtypeStruct((NPAD, D), jnp.float32)
    return pl.pallas_call(
        _proj_body,
        grid=(_NBLK,),
        in_specs=[pl.BlockSpec((_BN, din), lambda i: (i, 0)),
                  wspec, bspec, wspec, bspec, wspec, bspec, wspec, bspec],
        out_specs=[ospec, ospec, ospec, ospec],
        out_shape=[out_sh, out_sh, out_sh, out_sh],
    )(x, Wq, bq.reshape(1, D), Wk, bk.reshape(1, D),
      Wv, bv.reshape(1, D), Ws, bs.reshape(1, D))


_MESH = plsc.VectorSubcoreMesh(core_axis_name="c", subcore_axis_name="s")


@functools.partial(
    pl.kernel,
    out_type=(jax.ShapeDtypeStruct((E * H,), jnp.float32),
              jax.ShapeDtypeStruct((NW * 16,), jnp.float32)),
    mesh=_MESH,
    compiler_params=_SC_PARAMS,
    scratch_types=[
        pltpu.VMEM((2, CH), jnp.int32),
        pltpu.VMEM((2, CH), jnp.int32),
        pltpu.VMEM((2, CH, C), jnp.float32),
        pltpu.VMEM((2, CH, C), jnp.float32),
        pltpu.VMEM((HCH,), jnp.float32),
        pltpu.VMEM((16,), jnp.float32),
        pltpu.SemaphoreType.DMA,
        pltpu.SemaphoreType.DMA,
        pltpu.SemaphoreType.DMA,
    ],
)
def _alpha_kernel(q0, q1, q2, q3, k0, k1, k2, k3, src_hbm, dst_hbm,
                  alpha_hbm, tmax_hbm,
                  dstb, srcb, qrow, krow, abuf, maxref, semg0, semg1, semi):
    qs = (q0, q1, q2, q3)
    ks = (k0, k1, k2, k3)
    semg = (semg0, semg1)
    cid = lax.axis_index("c")
    sid = lax.axis_index("s")
    wid = cid * NS + sid
    ebase = wid * EPW
    maxref[...] = jnp.full((16,), -jnp.inf, jnp.float32)
    lanes = lax.iota(jnp.int32, 16)

    def head_step(ch, s, h):
        p = h % 2
        if h == 0:
            @pl.when(ch < NCHUNK - 1)
            def _():
                nb = ebase + (ch + 1) * CH
                pltpu.async_copy(dst_hbm.at[pl.ds(nb, CH)], dstb.at[1 - s],
                                 semi)
                pltpu.async_copy(src_hbm.at[pl.ds(nb, CH)], srcb.at[1 - s],
                                 semi)
        pltpu.make_async_copy(qs[h].at[dstb.at[s]], qrow.at[p], semg[p]).wait()
        pltpu.make_async_copy(ks[h].at[srcb.at[s]], krow.at[p], semg[p]).wait()
        if h < H - 1:
            pltpu.async_copy(qs[h + 1].at[dstb.at[s]], qrow.at[1 - p],
                             semg[1 - p])
            pltpu.async_copy(ks[h + 1].at[srcb.at[s]], krow.at[1 - p],
                             semg[1 - p])
        else:
            @pl.when(ch < NCHUNK - 1)
            def _():
                pltpu.make_async_copy(dst_hbm.at[pl.ds(0, CH)],
                                      dstb.at[1 - s], semi).wait()
                pltpu.make_async_copy(src_hbm.at[pl.ds(0, CH)],
                                      srcb.at[1 - s], semi).wait()
                pltpu.async_copy(qs[0].at[dstb.at[1 - s]], qrow.at[0],
                                 semg[0])
                pltpu.async_copy(ks[0].at[srcb.at[1 - s]], krow.at[0],
                                 semg[0])
        for g in range(G):
            eidx = lanes + g * 16

            def cbody(c0, acc, _eidx=eidx, _p=p):
                for cc in range(16):
                    col = jnp.full((16,), c0 * 16 + cc, jnp.int32)
                    qv = plsc.load_gather(qrow.at[_p], [_eidx, col])
                    kv = plsc.load_gather(krow.at[_p], [_eidx, col])
                    acc = acc + qv * kv
                return acc

            acc = lax.fori_loop(0, C // 16, cbody, jnp.zeros((16,), jnp.float32))
            abuf[pl.ds(h * CH + g * 16, 16)] = acc
            maxref[...] = jnp.maximum(maxref[...], acc)

    def chunk_body(ch, carry):
        s = ch & 1
        for h in range(H):
            head_step(ch, s, h)
        gc = wid * NCHUNK + ch
        pltpu.sync_copy(abuf, alpha_hbm.at[pl.ds(gc * HCH, HCH)])
        return carry

    pltpu.sync_copy(dst_hbm.at[pl.ds(ebase, CH)], dstb.at[0])
    pltpu.sync_copy(src_hbm.at[pl.ds(ebase, CH)], srcb.at[0])
    pltpu.async_copy(qs[0].at[dstb.at[0]], qrow.at[0], semg[0])
    pltpu.async_copy(ks[0].at[srcb.at[0]], krow.at[0], semg[0])

    lax.fori_loop(0, NCHUNK, chunk_body, 0)
    pltpu.sync_copy(maxref, tmax_hbm.at[pl.ds(wid * 16, 16)])


@functools.partial(
    pl.kernel,
    out_type=(jax.ShapeDtypeStruct((H, NC, NPAD, CHALF), jnp.float32),
              jax.ShapeDtypeStruct((H, NS, NPAD), jnp.float32)),
    mesh=_MESH,
    compiler_params=_SC_PARAMS,
    scratch_types=[
        pltpu.VMEM((2, CH), jnp.int32),
        pltpu.VMEM((2, CH), jnp.int32),
        pltpu.VMEM((2, CH), jnp.int32),
        pltpu.VMEM((2, CH), jnp.int32),
        pltpu.VMEM((2, CH), jnp.float32),
        pltpu.VMEM((2, CH, CHALF), jnp.float32),
        pltpu.VMEM((2, CH, CHALF), jnp.float32),
        pltpu.VMEM((NPAD,), jnp.float32),
        pltpu.VMEM((NW * 16,), jnp.float32),
        pltpu.VMEM_SHARED((NPAD, CHALF), jnp.float32),
        pltpu.SemaphoreType.DMA,
        pltpu.SemaphoreType.DMA,
        pltpu.SemaphoreType.DMA,
    ],
)
def _accum_kernel(vtab, alpha_hbm, src_hbm, dst_hbm, tmax_hbm,
                  znum_hbm, zden_hbm, num_out, den_out,
                  srcb, srcb2, dstb, dstsc, ab, vrow, wrow, dpriv, tmbuf,
                  spnum, semv, semi, semsc):
    cid = lax.axis_index("c")
    sid = lax.axis_index("s")
    ebase = sid * EPS
    gbase = sid * ACHUNK
    lanes = lax.iota(jnp.int32, 16)

    pltpu.sync_copy(tmax_hbm, tmbuf)
    m = tmbuf[pl.ds(0, 16)]
    for k in range(1, NW):
        m = jnp.maximum(m, tmbuf[pl.ds(k * 16, 16)])
    gmax = jnp.full((16,), jnp.max(m), jnp.float32)

    def head_body(hh, carry):
        def issue_idx(ch_next, s_next):
            nb = ebase + ch_next * CH
            ga = (gbase + ch_next) * HCH + hh * CH
            pltpu.async_copy(dst_hbm.at[pl.ds(nb, CH)], dstb.at[s_next], semi)
            pltpu.async_copy(src_hbm.at[pl.ds(nb, CH)], srcb.at[s_next], semi)
            pltpu.async_copy(alpha_hbm.at[pl.ds(ga, CH)], ab.at[s_next], semi)

        def wait_idx(s_next):
            pltpu.make_async_copy(dst_hbm.at[pl.ds(0, CH)], dstb.at[s_next],
                                  semi).wait()
            pltpu.make_async_copy(src_hbm.at[pl.ds(0, CH)], srcb.at[s_next],
                                  semi).wait()
            pltpu.make_async_copy(alpha_hbm.at[pl.ds(0, CH)], ab.at[s_next],
                                  semi).wait()

        def make_vidx(s):
            off = cid * 4 + hh
            for g in range(G):
                sl = pl.ds(g * 16, 16)
                srcb2[s, sl] = srcb[s, sl] * 8 + off

        def drain_scatter(s):
            pltpu.make_async_copy(wrow.at[s], spnum.at[dstsc.at[s]],
                                  semsc).wait()

        def chunk_body(ch, carry2):
            s = ch & 1
            @pl.when(ch < ACHUNK - 1)
            def _():
                issue_idx(ch + 1, 1 - s)
            pltpu.make_async_copy(vtab.at[srcb2.at[s]], vrow.at[s],
                                  semv).wait()
            @pl.when(ch >= 2)
            def _():
                drain_scatter(s)
            for g in range(G):
                sl = pl.ds(g * 16, 16)
                dstsc[s, sl] = dstb[s, sl]
            for g in range(G):
                eidx = lanes + g * 16
                a = ab[s, pl.ds(g * 16, 16)]
                w = jnp.exp(a - gmax)
                dvec = dstb[s, pl.ds(g * 16, 16)]
                plsc.addupdate_scatter(dpriv, [dvec], w)

                def cbody(c0, carry3, _eidx=eidx, _w=w, _s=s):
                    for cc in range(16):
                        col = jnp.full((16,), c0 * 16 + cc, jnp.int32)
                        vv = plsc.load_gather(vrow.at[_s], [_eidx, col])
                        plsc.store_scatter(wrow.at[_s], [_eidx, col],
                                           vv * _w)
                    return carry3

                lax.fori_loop(0, CHALF // 16, cbody, 0)
            @pl.when(ch < ACHUNK - 1)
            def _():
                wait_idx(1 - s)
                make_vidx(1 - s)
                pltpu.async_copy(vtab.at[srcb2.at[1 - s]], vrow.at[1 - s],
                                 semv)
            pltpu.async_copy(wrow.at[s], spnum.at[dstsc.at[s]], semsc,
                             add=True)
            return carry2

        pltpu.sync_copy(znum_hbm, spnum.at[pl.ds(sid * SLAB, SLAB)])
        pltpu.sync_copy(zden_hbm, dpriv)
        plsc.subcore_barrier()

        pltpu.sync_copy(dst_hbm.at[pl.ds(ebase, CH)], dstb.at[0])
        pltpu.sync_copy(src_hbm.at[pl.ds(ebase, CH)], srcb.at[0])
        pltpu.sync_copy(alpha_hbm.at[pl.ds(gbase * HCH + hh * CH, CH)],
                        ab.at[0])
        make_vidx(0)
        pltpu.async_copy(vtab.at[srcb2.at[0]], vrow.at[0], semv)

        lax.fori_loop(0, ACHUNK, chunk_body, 0)
        drain_scatter(0)
        drain_scatter(1)
        plsc.subcore_barrier()
        pltpu.sync_copy(spnum.at[pl.ds(sid * SLAB, SLAB)],
                        num_out.at[hh, cid, pl.ds(sid * SLAB, SLAB)])

        @pl.when(cid == 0)
        def _():
            pltpu.sync_copy(dpriv, den_out.at[hh, sid])
        return carry

    lax.fori_loop(0, H, head_body, 0)


def _merge_heads(num_ref, den_ref, s_ref):
    outs = []
    for h in range(H):
        n = jnp.concatenate([num_ref[h, 0], num_ref[h, 1]], axis=-1)
        d = jnp.sum(den_ref[h], axis=0)[:, None]
        safe = jnp.where(d > 0, d, 1.0)
        outs.append(jnp.where(d > 0, n / safe, 0.0))
    hh = jnp.concatenate(outs, axis=-1) + s_ref[...]
    return jnp.maximum(hh, 0.0)


def _combine_body(num_ref, den_ref, s_ref, o_ref):
    o_ref[...] = _merge_heads(num_ref, den_ref, s_ref)


def _combine_final_body(num_ref, den_ref, s_ref, wl_ref, bl_ref, o_ref):
    h = _merge_heads(num_ref, den_ref, s_ref)
    z = jnp.dot(h, wl_ref[...], preferred_element_type=jnp.float32) + bl_ref[...]
    o_ref[...] = jax.nn.sigmoid(z)


_NSPEC = pl.BlockSpec((H, NC, _BN, CHALF), lambda i: (0, 0, i, 0))
_DSPEC = pl.BlockSpec((H, NS, _BN), lambda i: (0, 0, i))
_SSPEC = pl.BlockSpec((_BN, D), lambda i: (i, 0))


def _combine(num, den, S):
    return pl.pallas_call(
        _combine_body,
        grid=(_NBLK,),
        in_specs=[_NSPEC, _DSPEC, _SSPEC],
        out_specs=_SSPEC,
        out_shape=jax.ShapeDtypeStruct((NPAD, D), jnp.float32),
    )(num, den, S)


def _combine_final(num, den, S, Wl, bl):
    return pl.pallas_call(
        _combine_final_body,
        grid=(_NBLK,),
        in_specs=[_NSPEC, _DSPEC, _SSPEC,
                  pl.BlockSpec((D, 1), lambda i: (0, 0)),
                  pl.BlockSpec((1, 1), lambda i: (0, 0))],
        out_specs=pl.BlockSpec((_BN, 1), lambda i: (i, 0)),
        out_shape=jax.ShapeDtypeStruct((NPAD, 1), jnp.float32),
    )(num, den, S, Wl, bl.reshape(1, 1))


def _attention_layer(xin, src, dst, znum, zden, Wq, bq, Wk, bk, Wv, bv, Ws, bs):
    Q, K, V, S = _project(xin, Wq, bq, Wk, bk, Wv, bv, Ws, bs)
    qh = [Q[:, h * C:(h + 1) * C] for h in range(H)]
    kh = [K[:, h * C:(h + 1) * C] for h in range(H)]
    vtab = V.reshape(NPAD, H, 2, CHALF).transpose(0, 2, 1, 3).reshape(
        NPAD * 2 * H, CHALF)
    alpha, tmax = _alpha_kernel(*qh, *kh, src, dst)
    num, den = _accum_kernel(vtab, alpha, src, dst, tmax, znum, zden)
    return num, den, S


def kernel(x, edge_index, edge_attr, Wq0, bq0, Wk0, bk0, Wv0, bv0, Ws0, bs0,
           Wq1, bq1, Wk1, bk1, Wv1, bv1, Ws1, bs1, Wl, bl):
    src = edge_index[0]
    dst = edge_index[1]
    znum = jnp.zeros((SLAB, CHALF), jnp.float32)
    zden = jnp.zeros((NPAD,), jnp.float32)

    xp = jnp.pad(x, ((0, NPAD - N), (0, 0)))
    num, den, S0 = _attention_layer(xp, src, dst, znum, zden,
                                    Wq0, bq0, Wk0, bk0, Wv0, bv0, Ws0, bs0)
    h0 = _combine(num, den, S0)
    num, den, S1 = _attention_layer(h0, src, dst, znum, zden,
                                    Wq1, bq1, Wk1, bk1, Wv1, bv1, Ws1, bs1)
    return _combine_final(num, den, S1, Wl, bl)[:N]
